# baseline (speedup 1.0000x reference)
import functools

import jax
import jax.numpy as jnp
from jax import lax
from jax.experimental import pallas as pl
from jax.experimental.pallas import tpu as pltpu
from jax.experimental.pallas import tpu_sc as plsc

N = 10000
E = 320000
D = 128

NC = 2
NS = 16
NW = NC * NS
EW = E // NW
C = 80
NCHUNK = EW // C
NB = 5
SUPER = NCHUNK // NB
G = C // 16
L = 16


NBLK = 25
BR = N // NBLK


def _cond_body(sx_ref, dv_ref, o_ref, acc_ref):
    i = pl.program_id(0)

    @pl.when(i == 0)
    def _():
        acc_ref[0] = jnp.int32(1)

    blk = jnp.all(jnp.abs(sx_ref[...] - dv_ref[...])
                  <= (1e-8 + 1e-5 * jnp.abs(dv_ref[...])))
    acc_ref[0] = acc_ref[0] * blk.astype(jnp.int32)

    @pl.when(i == NBLK - 1)
    def _():
        o_ref[0, 0] = acc_ref[0]


_cond_call = pl.pallas_call(
    _cond_body,
    grid=(NBLK,),
    in_specs=[
        pl.BlockSpec((BR, D), lambda i: (i, 0)),
        pl.BlockSpec((BR, D), lambda i: (i, 0)),
    ],
    out_specs=pl.BlockSpec(memory_space=pltpu.SMEM),
    out_shape=jax.ShapeDtypeStruct((1, 1), jnp.int32),
    scratch_shapes=[pltpu.SMEM((1,), jnp.int32)],
)


def _proj_body(cond_ref, sx_ref, dv_ref, w_ref, nd_ref, h_ref, hn_ref):
    cond = cond_ref[0, 0] == 1
    nd = jnp.where(cond, sx_ref[...], dv_ref[...])
    nd_ref[...] = nd
    h = jnp.dot(nd, w_ref[...], preferred_element_type=jnp.float32)
    h_ref[...] = h
    hn_ref[...] = -h


_proj_call = pl.pallas_call(
    _proj_body,
    grid=(NBLK,),
    in_specs=[
        pl.BlockSpec(memory_space=pltpu.SMEM),
        pl.BlockSpec((BR, D), lambda i: (i, 0)),
        pl.BlockSpec((BR, D), lambda i: (i, 0)),
        pl.BlockSpec((D, D), lambda i: (0, 0)),
    ],
    out_specs=(
        pl.BlockSpec((BR, D), lambda i: (i, 0)),
        pl.BlockSpec((BR, D), lambda i: (i, 0)),
        pl.BlockSpec((BR, D), lambda i: (i, 0)),
    ),
    out_shape=(
        jax.ShapeDtypeStruct((N, D), jnp.float32),
        jax.ShapeDtypeStruct((N, D), jnp.float32),
        jax.ShapeDtypeStruct((N, D), jnp.float32),
    ),
)


def _tc_call(sx, dv, w):
    cond = _cond_call(sx, dv)
    return _proj_call(cond, sx, dv, w)


@functools.cache
def _get_sc_scores():
    mesh = plsc.VectorSubcoreMesh(
        core_axis_name="c", subcore_axis_name="s",
        num_cores=NC, num_subcores=NS)

    @functools.partial(
        pl.kernel,
        out_type=jax.ShapeDtypeStruct((E,), jnp.float32),
        mesh=mesh,
        scratch_types=(
            [
                pltpu.VMEM((EW,), jnp.int32),
                pltpu.VMEM((EW,), jnp.int32),
                pltpu.VMEM((EW,), jnp.float32),
                pltpu.VMEM((L,), jnp.float32),
            ]
            + [pltpu.VMEM((C, D), jnp.float32)] * NB
            + [pltpu.SemaphoreType.DMA] * (2 * NB)
        ),
        compiler_params=pltpu.CompilerParams(needs_layout_passes=False),
    )
    def _sc_scores(h_hbm, hn_hbm, si_hbm, di_hbm, th_hbm, out_hbm,
                   si_v, di_v, out_v, th_v, *ring):
        bufs = ring[:NB]
        sg = ring[NB:2 * NB]
        sa = ring[2 * NB:3 * NB]
        wid = lax.axis_index("s") * NC + lax.axis_index("c")
        base0 = wid * EW
        pltpu.sync_copy(si_hbm.at[pl.ds(base0, EW)], si_v)
        pltpu.sync_copy(di_hbm.at[pl.ds(base0, EW)], di_v)
        pltpu.sync_copy(th_hbm, th_v)
        th = th_v[...]
        lane = lax.iota(jnp.int32, L)

        def start_g(c, slot):
            pltpu.async_copy(
                h_hbm.at[si_v.at[pl.ds(c * C, C)]], bufs[slot], sg[slot],
                add=True)

        def wait_g(slot):
            pltpu.make_async_copy(
                h_hbm.at[si_v.at[pl.ds(0, C)]], bufs[slot], sg[slot]).wait()

        def start_a(c, slot):
            pltpu.async_copy(
                hn_hbm.at[di_v.at[pl.ds(c * C, C)]], bufs[slot], sa[slot],
                add=True)

        def wait_a(slot):
            pltpu.make_async_copy(
                hn_hbm.at[di_v.at[pl.ds(0, C)]], bufs[slot], sa[slot]).wait()

        zvec = jnp.zeros((L,), jnp.float32)

        def compute_chunk(buf, c):
            def group_body(g, carry2):
                res = jnp.zeros((L,), jnp.float32)
                for e in range(L):
                    row = g * L + e
                    acc0 = jnp.zeros((L,), jnp.float32)
                    acc1 = jnp.zeros((L,), jnp.float32)
                    for kk in range(D // L // 2):
                        v0 = buf[row, pl.ds((2 * kk) * L, L)]
                        buf[row, pl.ds((2 * kk) * L, L)] = zvec
                        v1 = buf[row, pl.ds((2 * kk + 1) * L, L)]
                        buf[row, pl.ds((2 * kk + 1) * L, L)] = zvec
                        acc0 = acc0 + v0 * v0
                        acc1 = acc1 + v1 * v1
                    tot = jnp.sum(acc0 + acc1)
                    res = jnp.where(lane == e, tot, res)
                out_v[pl.ds(c * C + g * L, L)] = 1.0 / (1.0 + jnp.exp(res - th))
                return carry2

            lax.fori_loop(0, G, group_body, 0)

        def zero_body(r, carry2):
            for bb in range(NB):
                for kk in range(D // L):
                    bufs[bb][r, pl.ds(kk * L, L)] = zvec
            return carry2

        lax.fori_loop(0, C, zero_body, 0)

        start_g(0, 0)
        start_a(0, 0)
        start_g(1, 1)
        start_a(1, 1)
        start_g(2, 2)
        start_a(2, 2)

        def super_body(s, carry):
            for b in range(NB):
                c = s * NB + b

                @pl.when(c + 3 < NCHUNK)
                def _():
                    start_g(c + 3, (b + 3) % NB)
                    start_a(c + 3, (b + 3) % NB)

                wait_g(b)
                wait_a(b)
                compute_chunk(bufs[b], c)
            return carry

        lax.fori_loop(0, SUPER, super_body, 0)
        pltpu.sync_copy(out_v, out_hbm.at[pl.ds(base0, EW)])

    return _sc_scores


def kernel(stored_x, dynamicVariable, edge_index, W, theta):
    ndata, h, hn = _tc_call(stored_x, dynamicVariable, W)
    si = edge_index[0].astype(jnp.int32)
    di = edge_index[1].astype(jnp.int32)
    th_arr = jnp.broadcast_to(theta.astype(jnp.float32), (L,))
    scores = _get_sc_scores()(h, hn, si, di, th_arr)
    return ndata, scores


# speedup vs baseline: 1.0446x; 1.0446x over previous
import functools

import jax
import jax.numpy as jnp
from jax import lax
from jax.experimental import pallas as pl
from jax.experimental.pallas import tpu as pltpu
from jax.experimental.pallas import tpu_sc as plsc

N = 10000
E = 320000
D = 128

NC = 2
NS = 16
NW = NC * NS
EW = E // NW
C = 400
NCHUNK = EW // C
NB = 2
G = C // 16
L = 16


def _tc_body(sx_ref, dv_ref, w_ref, nd_ref, h_ref, hn_ref):
    sx = sx_ref[...]
    dv = dv_ref[...]
    cond = jnp.all(jnp.abs(sx - dv) <= (1e-8 + 1e-5 * jnp.abs(dv)))
    nd = jnp.where(cond, sx, dv)
    nd_ref[...] = nd
    h = jnp.dot(nd, w_ref[...], preferred_element_type=jnp.float32)
    h_ref[...] = h
    hn_ref[...] = -h


_tc_call = pl.pallas_call(
    _tc_body,
    out_shape=(
        jax.ShapeDtypeStruct((N, D), jnp.float32),
        jax.ShapeDtypeStruct((N, D), jnp.float32),
        jax.ShapeDtypeStruct((N, D), jnp.float32),
    ),
)


@functools.cache
def _get_sc_scores():
    mesh = plsc.VectorSubcoreMesh(
        core_axis_name="c", subcore_axis_name="s",
        num_cores=NC, num_subcores=NS)

    @functools.partial(
        pl.kernel,
        out_type=jax.ShapeDtypeStruct((E,), jnp.float32),
        mesh=mesh,
        scratch_types=(
            [
                pltpu.VMEM((EW,), jnp.int32),
                pltpu.VMEM((EW,), jnp.int32),
                pltpu.VMEM((L,), jnp.float32),
            ]
            + [pltpu.VMEM((C, D), jnp.float32)] * NB
            + [pltpu.VMEM((C,), jnp.float32)] * NB
            + [pltpu.SemaphoreType.DMA] * (3 * NB)
        ),
        compiler_params=pltpu.CompilerParams(needs_layout_passes=False),
    )
    def _sc_scores(h_hbm, hn_hbm, si_hbm, di_hbm, th_hbm, out_hbm,
                   si_v, di_v, th_v, *ring):
        bufs = ring[:NB]
        outs = ring[NB:2 * NB]
        sg = ring[2 * NB:3 * NB]
        sa = ring[3 * NB:4 * NB]
        so = ring[4 * NB:5 * NB]
        wid = lax.axis_index("s") * NC + lax.axis_index("c")
        base0 = wid * EW
        pltpu.sync_copy(si_hbm.at[pl.ds(base0, EW)], si_v)
        pltpu.sync_copy(di_hbm.at[pl.ds(base0, EW)], di_v)
        pltpu.sync_copy(th_hbm, th_v)
        th = th_v[...]
        lane = lax.iota(jnp.int32, L)
        zvec = jnp.zeros((L,), jnp.float32)

        def start_pair(c, slot):
            pltpu.async_copy(
                h_hbm.at[si_v.at[pl.ds(c * C, C)]], bufs[slot], sg[slot],
                add=True)
            pltpu.async_copy(
                hn_hbm.at[di_v.at[pl.ds(c * C, C)]], bufs[slot], sa[slot],
                add=True)

        def wait_pair(slot):
            pltpu.make_async_copy(
                h_hbm.at[si_v.at[pl.ds(0, C)]], bufs[slot], sg[slot]).wait()
            pltpu.make_async_copy(
                hn_hbm.at[di_v.at[pl.ds(0, C)]], bufs[slot], sa[slot]).wait()

        def start_out(c, slot):
            pltpu.async_copy(
                outs[slot], out_hbm.at[pl.ds(base0 + c * C, C)], so[slot])

        def wait_out(slot):
            pltpu.make_async_copy(
                outs[slot], out_hbm.at[pl.ds(base0, C)], so[slot]).wait()

        def compute_chunk(buf, out_b):
            def group_body(g, carry2):
                res = jnp.zeros((L,), jnp.float32)
                for e in range(L):
                    row = g * L + e
                    acc0 = jnp.zeros((L,), jnp.float32)
                    acc1 = jnp.zeros((L,), jnp.float32)
                    for kk in range(D // L // 2):
                        v0 = buf[row, pl.ds((2 * kk) * L, L)]
                        buf[row, pl.ds((2 * kk) * L, L)] = zvec
                        v1 = buf[row, pl.ds((2 * kk + 1) * L, L)]
                        buf[row, pl.ds((2 * kk + 1) * L, L)] = zvec
                        acc0 = acc0 + v0 * v0
                        acc1 = acc1 + v1 * v1
                    tot = jnp.sum(acc0 + acc1)
                    res = jnp.where(lane == e, tot, res)
                out_b[pl.ds(g * L, L)] = 1.0 / (1.0 + jnp.exp(res - th))
                return carry2

            lax.fori_loop(0, G, group_body, 0)

        def zero_body(r, carry2):
            for bb in range(NB):
                for kk in range(D // L):
                    bufs[bb][r, pl.ds(kk * L, L)] = zvec
            return carry2

        lax.fori_loop(0, C, zero_body, 0)

        start_pair(0, 0)

        def super_body(s, carry):
            for b in range(NB):
                c = s * NB + b
                wait_pair(b)
                start_pair(c + 1, 1 - b)

                @pl.when(c >= NB)
                def _():
                    wait_out(b)

                compute_chunk(bufs[b], outs[b])
                start_out(c, b)
            return carry

        lax.fori_loop(0, (NCHUNK - 1) // NB, super_body, 0)
        wait_pair(0)
        wait_out(0)
        compute_chunk(bufs[0], outs[0])
        pltpu.sync_copy(outs[0], out_hbm.at[pl.ds(base0 + (NCHUNK - 1) * C, C)])
        wait_out(1)

    return _sc_scores


def kernel(stored_x, dynamicVariable, edge_index, W, theta):
    ndata, h, hn = _tc_call(stored_x, dynamicVariable, W)
    si = edge_index[0].astype(jnp.int32)
    di = edge_index[1].astype(jnp.int32)
    th_arr = jnp.broadcast_to(theta.astype(jnp.float32), (L,))
    scores = _get_sc_scores()(h, hn, si, di, th_arr)
    return ndata, scores


# speedup vs baseline: 1.1556x; 1.1062x over previous
import functools

import jax
import jax.numpy as jnp
from jax import lax
from jax.experimental import pallas as pl
from jax.experimental.pallas import tpu as pltpu
from jax.experimental.pallas import tpu_sc as plsc

N = 10000
E = 320000
D = 128

NC = 2
NS = 16
NW = NC * NS
EW = E // NW
C = 80
NCHUNK = EW // C
NB = 5
SUPER = NCHUNK // NB
G = C // 16
L = 16


def _tc_body(sx_ref, dv_ref, w_ref, nd_ref, h_ref, hn_ref):
    sx = sx_ref[...]
    dv = dv_ref[...]
    cond = jnp.all(jnp.abs(sx - dv) <= (1e-8 + 1e-5 * jnp.abs(dv)))
    nd = jnp.where(cond, sx, dv)
    nd_ref[...] = nd
    h = jnp.dot(nd, w_ref[...], preferred_element_type=jnp.float32)
    h_ref[...] = h
    hn_ref[...] = -h


_tc_call = pl.pallas_call(
    _tc_body,
    out_shape=(
        jax.ShapeDtypeStruct((N, D), jnp.float32),
        jax.ShapeDtypeStruct((N, D), jnp.float32),
        jax.ShapeDtypeStruct((N, D), jnp.float32),
    ),
)


@functools.cache
def _get_sc_scores():
    mesh = plsc.VectorSubcoreMesh(
        core_axis_name="c", subcore_axis_name="s",
        num_cores=NC, num_subcores=NS)

    @functools.partial(
        pl.kernel,
        out_type=jax.ShapeDtypeStruct((E,), jnp.float32),
        mesh=mesh,
        scratch_types=(
            [
                pltpu.VMEM((EW,), jnp.int32),
                pltpu.VMEM((EW,), jnp.int32),
                pltpu.VMEM((EW,), jnp.float32),
                pltpu.VMEM((L,), jnp.float32),
            ]
            + [pltpu.VMEM((C, D), jnp.float32)] * NB
            + [pltpu.SemaphoreType.DMA] * (2 * NB)
        ),
        compiler_params=pltpu.CompilerParams(needs_layout_passes=False),
    )
    def _sc_scores(h_hbm, hn_hbm, si_hbm, di_hbm, th_hbm, out_hbm,
                   si_v, di_v, out_v, th_v, *ring):
        bufs = ring[:NB]
        sg = ring[NB:2 * NB]
        sa = ring[2 * NB:3 * NB]
        wid = lax.axis_index("s") * NC + lax.axis_index("c")
        base0 = wid * EW
        pltpu.sync_copy(si_hbm.at[pl.ds(base0, EW)], si_v)
        pltpu.sync_copy(di_hbm.at[pl.ds(base0, EW)], di_v)
        pltpu.sync_copy(th_hbm, th_v)
        th = th_v[...]
        lane = lax.iota(jnp.int32, L)

        def start_g(c, slot):
            pltpu.async_copy(
                h_hbm.at[si_v.at[pl.ds(c * C, C)]], bufs[slot], sg[slot],
                add=True)

        def wait_g(slot):
            pltpu.make_async_copy(
                h_hbm.at[si_v.at[pl.ds(0, C)]], bufs[slot], sg[slot]).wait()

        def start_a(c, slot):
            pltpu.async_copy(
                hn_hbm.at[di_v.at[pl.ds(c * C, C)]], bufs[slot], sa[slot],
                add=True)

        def wait_a(slot):
            pltpu.make_async_copy(
                hn_hbm.at[di_v.at[pl.ds(0, C)]], bufs[slot], sa[slot]).wait()

        zvec = jnp.zeros((L,), jnp.float32)

        def compute_chunk(buf, c):
            def group_body(g, carry2):
                res = jnp.zeros((L,), jnp.float32)
                for e in range(L):
                    row = g * L + e
                    acc0 = jnp.zeros((L,), jnp.float32)
                    acc1 = jnp.zeros((L,), jnp.float32)
                    for kk in range(D // L // 2):
                        v0 = buf[row, pl.ds((2 * kk) * L, L)]
                        buf[row, pl.ds((2 * kk) * L, L)] = zvec
                        v1 = buf[row, pl.ds((2 * kk + 1) * L, L)]
                        buf[row, pl.ds((2 * kk + 1) * L, L)] = zvec
                        acc0 = acc0 + v0 * v0
                        acc1 = acc1 + v1 * v1
                    tot = jnp.sum(acc0 + acc1)
                    res = jnp.where(lane == e, tot, res)
                out_v[pl.ds(c * C + g * L, L)] = 1.0 / (1.0 + jnp.exp(res - th))
                return carry2

            lax.fori_loop(0, G, group_body, 0)

        def zero_body(r, carry2):
            for bb in range(NB):
                for kk in range(D // L):
                    bufs[bb][r, pl.ds(kk * L, L)] = zvec
            return carry2

        lax.fori_loop(0, C, zero_body, 0)

        start_g(0, 0)
        start_a(0, 0)
        start_g(1, 1)
        start_a(1, 1)
        start_g(2, 2)
        start_a(2, 2)

        def super_body(s, carry):
            for b in range(NB):
                c = s * NB + b

                @pl.when(c + 3 < NCHUNK)
                def _():
                    start_g(c + 3, (b + 3) % NB)
                    start_a(c + 3, (b + 3) % NB)

                wait_g(b)
                wait_a(b)
                compute_chunk(bufs[b], c)
            return carry

        lax.fori_loop(0, SUPER, super_body, 0)
        pltpu.sync_copy(out_v, out_hbm.at[pl.ds(base0, EW)])

    return _sc_scores


def kernel(stored_x, dynamicVariable, edge_index, W, theta):
    ndata, h, hn = _tc_call(stored_x, dynamicVariable, W)
    si = edge_index[0].astype(jnp.int32)
    di = edge_index[1].astype(jnp.int32)
    th_arr = jnp.broadcast_to(theta.astype(jnp.float32), (L,))
    scores = _get_sc_scores()(h, hn, si, di, th_arr)
    return ndata, scores
